# baseline (speedup 1.0000x reference)
import functools

import jax
import jax.numpy as jnp
from jax import lax
from jax.experimental import pallas as pl
from jax.experimental.pallas import tpu as pltpu
from jax.experimental.pallas import tpu_sc as plsc

B = 4096
VOCAB = 1000
DIM = 128
NCOLS = 26

_info = plsc.get_sparse_core_info()
_NC = _info.num_cores
_NS = _info.num_subcores
_NW = _NC * _NS
_CPC = NCOLS // _NC
_RPT = B // _NS
_NSPLIT = 2
_SEG = _RPT // _NSPLIT
_NTASK = _CPC * _NSPLIT
_NBUF = 3
_HBM_COLS = 5
_NSTAGE = _CPC - _HBM_COLS

_mesh = plsc.VectorSubcoreMesh(core_axis_name="c", subcore_axis_name="s")


@functools.partial(
    pl.kernel,
    mesh=_mesh,
    out_type=[jax.ShapeDtypeStruct((B, DIM), jnp.float32)] * NCOLS,
    scratch_types=(
        [pltpu.VMEM((_CPC, _NSPLIT, _SEG), jnp.int32)]
        + [pltpu.VMEM((_SEG, DIM), jnp.float32)] * _NBUF
        + [pltpu.VMEM_SHARED((_NSTAGE * VOCAB, DIM), jnp.float32)]
        + [pltpu.SemaphoreType.DMA] * (1 + 2 * _NBUF)
    ),
)
def _gather_all(*refs):
    idx_hbm = refs[0]
    tab_refs = refs[1:1 + NCOLS]
    out_refs = refs[1 + NCOLS:1 + 2 * NCOLS]
    scratch = refs[1 + 2 * NCOLS:]
    idx_all = scratch[0]
    rows = scratch[1:1 + _NBUF]
    spmem_tab = scratch[1 + _NBUF]
    sem_g = scratch[3 + _NBUF:3 + 2 * _NBUF]
    sem_o = scratch[3 + 2 * _NBUF:3 + 3 * _NBUF]
    sem_st = scratch[2 + _NBUF]
    cid = lax.axis_index("c")
    sid = lax.axis_index("s")
    wid = sid * _NC + cid
    pltpu.sync_copy(idx_hbm.at[wid], idx_all)
    st_cp = {}
    for half in range(_NC):
        for t in range(_HBM_COLS, _CPC):
            @pl.when((cid == half) & (sid == t))
            def _(half=half, t=t):
                st_cp[(half, t)] = [pltpu.async_copy(
                    tab_refs[half * _CPC + t],
                    spmem_tab.at[pl.ds((t - _HBM_COLS) * VOCAB, VOCAB)],
                    sem_st)]

    def run(cols, tasks, from_spmem):
        n = len(tasks)
        g = [None] * n
        o = [None] * n

        def start_gather(k):
            t, j = tasks[k]
            b = k % _NBUF
            if from_spmem:
                src = spmem_tab.at[idx_all.at[t, j]]
            else:
                src = tab_refs[cols[t]].at[idx_all.at[t, j]]
            g[k] = pltpu.async_copy(src, rows[b], sem_g[b])

        for k in range(min(_NBUF - 1, n)):
            start_gather(k)
        for k in range(n):
            b = k % _NBUF
            t, j = tasks[k]
            if k + _NBUF - 1 < n:
                if k >= 1:
                    o[k - 1].wait()
                start_gather(k + _NBUF - 1)
            g[k].wait()
            base = sid * _RPT + j * _SEG
            o[k] = pltpu.async_copy(rows[b],
                                    out_refs[cols[t]].at[pl.ds(base, _SEG)],
                                    sem_o[b])
        for k in range(max(0, n - _NBUF), n):
            o[k].wait()

    pre = [(t, j) for t in range(_HBM_COLS) for j in range(_NSPLIT)]
    post = [(t, j) for t in range(_HBM_COLS, _CPC) for j in range(_NSPLIT)]
    groups = [list(range(_CPC * h, _CPC * (h + 1))) for h in range(_NC)]
    for half in range(_NC):
        @pl.when(cid == half)
        def _(half=half):
            run(groups[half], pre, from_spmem=False)
    for half in range(_NC):
        for t in range(_HBM_COLS, _CPC):
            @pl.when((cid == half) & (sid == t))
            def _(half=half, t=t):
                for cp in st_cp[(half, t)]:
                    cp.wait()
    plsc.subcore_barrier()
    for half in range(_NC):
        @pl.when(cid == half)
        def _(half=half):
            run(groups[half], post, from_spmem=True)


def kernel(cat0, cat1, cat2, cat3, cat4, cat5, cat6, cat7, cat8, cat9,
           cat10, cat11, cat12, cat13, cat14, cat15, cat16, cat17, cat18,
           cat19, cat20, cat21, cat22, cat23, cat24, cat25,
           table_cat0, table_cat1, table_cat2, table_cat3, table_cat4,
           table_cat5, table_cat6, table_cat7, table_cat8, table_cat9,
           table_cat10, table_cat11, table_cat12, table_cat13, table_cat14,
           table_cat15, table_cat16, table_cat17, table_cat18, table_cat19,
           table_cat20, table_cat21, table_cat22, table_cat23, table_cat24,
           table_cat25):
    cats = (cat0, cat1, cat2, cat3, cat4, cat5, cat6, cat7, cat8, cat9,
            cat10, cat11, cat12, cat13, cat14, cat15, cat16, cat17, cat18,
            cat19, cat20, cat21, cat22, cat23, cat24, cat25)
    tabs = (table_cat0, table_cat1, table_cat2, table_cat3, table_cat4,
            table_cat5, table_cat6, table_cat7, table_cat8, table_cat9,
            table_cat10, table_cat11, table_cat12, table_cat13, table_cat14,
            table_cat15, table_cat16, table_cat17, table_cat18, table_cat19,
            table_cat20, table_cat21, table_cat22, table_cat23, table_cat24,
            table_cat25)
    idx_cat = jnp.concatenate([c.reshape(1, B) for c in cats], axis=0)
    idx_5d = idx_cat.reshape(_NC, _CPC, _NS, _NSPLIT, _SEG)
    t_ar = jnp.arange(_CPC, dtype=jnp.int32)
    col_off = jnp.where(t_ar < _HBM_COLS, 0, (t_ar - _HBM_COLS) * VOCAB)
    idx_5d = idx_5d + col_off[None, :, None, None, None]
    idx_w = idx_5d.transpose(2, 0, 1, 3, 4).reshape(_NW, _CPC, _NSPLIT, _SEG)
    outs = _gather_all(idx_w, *tabs)
    return tuple(o.reshape(B, 1, DIM) for o in outs)


# speedup vs baseline: 1.0484x; 1.0484x over previous
import functools

import jax
import jax.numpy as jnp
from jax import lax
from jax.experimental import pallas as pl
from jax.experimental.pallas import tpu as pltpu
from jax.experimental.pallas import tpu_sc as plsc

B = 4096
VOCAB = 1000
DIM = 128
NCOLS = 26

_info = plsc.get_sparse_core_info()
_NC = _info.num_cores
_NS = _info.num_subcores
_NW = _NC * _NS
_CPC = NCOLS // _NC
_RPT = B // _NS
_NSPLIT = 2
_SEG = _RPT // _NSPLIT
_NTASK = _CPC * _NSPLIT
_NBUF = 3
_HBM_COLS = 4
_NSTAGE = _CPC - _HBM_COLS

_mesh = plsc.VectorSubcoreMesh(core_axis_name="c", subcore_axis_name="s")


@functools.partial(
    pl.kernel,
    mesh=_mesh,
    out_type=[jax.ShapeDtypeStruct((B, DIM), jnp.float32)] * NCOLS,
    scratch_types=(
        [pltpu.VMEM((_CPC, _NSPLIT, _SEG), jnp.int32)]
        + [pltpu.VMEM((_SEG, DIM), jnp.float32)] * _NBUF
        + [pltpu.VMEM_SHARED((_NSTAGE * VOCAB, DIM), jnp.float32)]
        + [pltpu.SemaphoreType.DMA] * (1 + 2 * _NBUF)
    ),
)
def _gather_all(*refs):
    idx_hbm = refs[0]
    tab_refs = refs[1:1 + NCOLS]
    out_refs = refs[1 + NCOLS:1 + 2 * NCOLS]
    scratch = refs[1 + 2 * NCOLS:]
    idx_all = scratch[0]
    rows = scratch[1:1 + _NBUF]
    spmem_tab = scratch[1 + _NBUF]
    sem_g = scratch[3 + _NBUF:3 + 2 * _NBUF]
    sem_o = scratch[3 + 2 * _NBUF:3 + 3 * _NBUF]
    sem_st = scratch[2 + _NBUF]
    cid = lax.axis_index("c")
    sid = lax.axis_index("s")
    wid = sid * _NC + cid
    pltpu.sync_copy(idx_hbm.at[wid], idx_all)
    st_cp = {}
    for half in range(_NC):
        for t in range(_HBM_COLS, _CPC):
            @pl.when((cid == half) & (sid == t))
            def _(half=half, t=t):
                st_cp[(half, t)] = [pltpu.async_copy(
                    tab_refs[half * _CPC + t],
                    spmem_tab.at[pl.ds((t - _HBM_COLS) * VOCAB, VOCAB)],
                    sem_st)]

    npre = _HBM_COLS * _NSPLIT

    def run(half):
        cols = list(range(_CPC * half, _CPC * (half + 1)))
        tasks = [(t, j) for t in range(_CPC) for j in range(_NSPLIT)]
        n = len(tasks)
        g = [None] * n
        o = [None] * n

        def start_gather(k):
            t, j = tasks[k]
            b = k % _NBUF
            if k >= npre:
                src = spmem_tab.at[idx_all.at[t, j]]
            else:
                src = tab_refs[cols[t]].at[idx_all.at[t, j]]
            g[k] = pltpu.async_copy(src, rows[b], sem_g[b])

        def barrier_if_needed(k):
            if k == npre:
                for t in range(_HBM_COLS, _CPC):
                    @pl.when(sid == t)
                    def _(t=t):
                        for cp in st_cp[(half, t)]:
                            cp.wait()
                plsc.subcore_barrier()

        for k in range(min(_NBUF - 1, n)):
            barrier_if_needed(k)
            start_gather(k)
        for k in range(n):
            b = k % _NBUF
            t, j = tasks[k]
            if k + _NBUF - 1 < n:
                if k >= 1:
                    o[k - 1].wait()
                barrier_if_needed(k + _NBUF - 1)
                start_gather(k + _NBUF - 1)
            g[k].wait()
            base = sid * _RPT + j * _SEG
            o[k] = pltpu.async_copy(rows[b],
                                    out_refs[cols[t]].at[pl.ds(base, _SEG)],
                                    sem_o[b])
        for k in range(max(0, n - _NBUF), n):
            o[k].wait()

    for half in range(_NC):
        @pl.when(cid == half)
        def _(half=half):
            run(half)


def kernel(cat0, cat1, cat2, cat3, cat4, cat5, cat6, cat7, cat8, cat9,
           cat10, cat11, cat12, cat13, cat14, cat15, cat16, cat17, cat18,
           cat19, cat20, cat21, cat22, cat23, cat24, cat25,
           table_cat0, table_cat1, table_cat2, table_cat3, table_cat4,
           table_cat5, table_cat6, table_cat7, table_cat8, table_cat9,
           table_cat10, table_cat11, table_cat12, table_cat13, table_cat14,
           table_cat15, table_cat16, table_cat17, table_cat18, table_cat19,
           table_cat20, table_cat21, table_cat22, table_cat23, table_cat24,
           table_cat25):
    cats = (cat0, cat1, cat2, cat3, cat4, cat5, cat6, cat7, cat8, cat9,
            cat10, cat11, cat12, cat13, cat14, cat15, cat16, cat17, cat18,
            cat19, cat20, cat21, cat22, cat23, cat24, cat25)
    tabs = (table_cat0, table_cat1, table_cat2, table_cat3, table_cat4,
            table_cat5, table_cat6, table_cat7, table_cat8, table_cat9,
            table_cat10, table_cat11, table_cat12, table_cat13, table_cat14,
            table_cat15, table_cat16, table_cat17, table_cat18, table_cat19,
            table_cat20, table_cat21, table_cat22, table_cat23, table_cat24,
            table_cat25)
    idx_cat = jnp.concatenate([c.reshape(1, B) for c in cats], axis=0)
    idx_5d = idx_cat.reshape(_NC, _CPC, _NS, _NSPLIT, _SEG)
    t_ar = jnp.arange(_CPC, dtype=jnp.int32)
    col_off = jnp.where(t_ar < _HBM_COLS, 0, (t_ar - _HBM_COLS) * VOCAB)
    idx_5d = idx_5d + col_off[None, :, None, None, None]
    idx_w = idx_5d.transpose(2, 0, 1, 3, 4).reshape(_NW, _CPC, _NSPLIT, _SEG)
    outs = _gather_all(idx_w, *tabs)
    return tuple(o.reshape(B, 1, DIM) for o in outs)
